# baseline (speedup 1.0000x reference)
import functools

import jax
import jax.numpy as jnp
from jax import lax
from jax.experimental import pallas as pl
from jax.experimental.pallas import tpu as pltpu
from jax.experimental.pallas import tpu_sc as plsc

_BATCH = 4
_SEQ = 8192
_D = 768
_N = _BATCH * _SEQ

_NC = 2
_NS = 16
_NW = _NC * _NS
_POS_PER_W = _SEQ // _NW
_CHUNK = 16
_GROUPS = _D // 16
_STEPS = _BATCH * _POS_PER_W // _CHUNK
_OUTER = _STEPS // 8


def _make_emb_kernel():
  mesh = plsc.VectorSubcoreMesh(core_axis_name="c", subcore_axis_name="s")

  @functools.partial(
      pl.kernel,
      out_type=jax.ShapeDtypeStruct((_N, _D), jnp.float32),
      mesh=mesh,
      scratch_types=[
          pltpu.VMEM((_BATCH * _POS_PER_W,), jnp.int32),
          pltpu.VMEM((_CHUNK, _D), jnp.float32),
          pltpu.VMEM((_CHUNK, _D), jnp.float32),
          pltpu.VMEM((_CHUNK, _D), jnp.float32),
          pltpu.VMEM((_CHUNK, _D), jnp.float32),
          pltpu.VMEM((_CHUNK, _D), jnp.float32),
          pltpu.VMEM((_CHUNK, _D), jnp.float32),
          pltpu.SemaphoreType.DMA,
          pltpu.SemaphoreType.DMA,
          pltpu.SemaphoreType.DMA,
      ],
  )
  def emb(idx_hbm, table_hbm, pos_hbm, out_hbm, idx_v, tok0, tok1, tok2, tok3,
          pos0, pos1, sem_g, sem_p, sem_w):
    cid = lax.axis_index("c")
    sid = lax.axis_index("s")
    wid = sid * _NC + cid
    pstart = wid * _POS_PER_W
    tb = (tok0, tok1, tok2, tok3)
    pb = (pos0, pos1)

    def start_gather(b, pc, buf):
      pltpu.async_copy(
          table_hbm.at[
              idx_v.at[pl.ds(b * _POS_PER_W + pc * _CHUNK, _CHUNK)]
          ],
          tb[buf],
          sem_g,
      )

    def wait_gather(buf):
      pltpu.make_async_copy(
          table_hbm.at[idx_v.at[pl.ds(0, _CHUNK)]], tb[buf], sem_g
      ).wait()

    def start_pos(pc, buf):
      pltpu.async_copy(
          pos_hbm.at[pl.ds(pstart + pc * _CHUNK, _CHUNK)], pb[buf], sem_p
      )

    def wait_pos(buf):
      pltpu.make_async_copy(
          pos_hbm.at[pl.ds(0, _CHUNK)], pb[buf], sem_p
      ).wait()

    def start_write(b, pc, buf):
      pltpu.async_copy(
          tb[buf],
          out_hbm.at[pl.ds(b * _SEQ + pstart + pc * _CHUNK, _CHUNK)],
          sem_w,
      )

    def wait_write(buf):
      pltpu.make_async_copy(
          tb[buf], out_hbm.at[pl.ds(0, _CHUNK)], sem_w
      ).wait()

    def add_rows(p, q, lo, hi):
      tok = tb[p]
      pos = pb[q]

      def rows2(i, c2):
        r = i * 2
        for r2 in range(2):
          for g in range(_GROUPS):
            plsc.addupdate(
                tok.at[r + r2, pl.ds(g * 16, 16)],
                pos[r + r2, pl.ds(g * 16, 16)],
            )
        return c2

      lax.fori_loop(lo // 2, hi // 2, rows2, 0)

    for b in range(_BATCH):
      pltpu.sync_copy(
          idx_hbm.at[pl.ds(b * _SEQ + pstart, _POS_PER_W)],
          idx_v.at[pl.ds(b * _POS_PER_W, _POS_PER_W)],
      )

    start_pos(0, 0)
    start_gather(0, 0, 0)
    start_gather(1, 0, 1)

    def outer(it, carry):
      pcb = it * 2
      for k in range(8):
        b = k % 4
        p = k % 4
        q = k // 4
        pc = pcb + (k // 4)
        wait_gather(p)
        if b == 0:
          wait_pos(q)
        add_rows(p, q, 0, _CHUNK // 2)
        if k < 2:
          @pl.when(it > 0)
          def _():
            wait_write((k + 2) % 4)
        else:
          wait_write((k + 2) % 4)
        if k < 6:
          start_gather((k + 2) % 4, pcb + (k + 2) // 4, (k + 2) % 4)
        else:
          @pl.when(it < _OUTER - 1)
          def _():
            start_gather((k + 2) % 4, pcb + 2, (k + 2) % 4)
        if k == 2:
          start_pos(pcb + 1, 1)
        elif k == 6:
          @pl.when(it < _OUTER - 1)
          def _():
            start_pos(pcb + 2, 0)
        add_rows(p, q, _CHUNK // 2, _CHUNK)
        start_write(b, pc, p)
      return carry

    lax.fori_loop(0, _OUTER, outer, 0)
    wait_write(2)
    wait_write(3)

  return emb


_emb = _make_emb_kernel()


@jax.jit
def kernel(x, tok_table, pos_table):
  idx = x.reshape(-1)
  out = _emb(idx, tok_table, pos_table)
  return out.reshape(x.shape[0], x.shape[1], _D)


# speedup vs baseline: 1.5980x; 1.5980x over previous
import functools

import jax
import jax.numpy as jnp
from jax import lax
from jax.experimental import pallas as pl
from jax.experimental.pallas import tpu as pltpu
from jax.experimental.pallas import tpu_sc as plsc

_BATCH = 4
_SEQ = 8192
_D = 768
_N = _BATCH * _SEQ

_NC = 2
_NS = 16
_NW = _NC * _NS
_POS_PER_W = _SEQ // _NW
_CHUNK = 16
_GROUPS = _D // 16
_STEPS = _BATCH * _POS_PER_W // _CHUNK
_OUTER = _STEPS // 8


def _make_emb_kernel():
  mesh = plsc.VectorSubcoreMesh(core_axis_name="c", subcore_axis_name="s")

  @functools.partial(
      pl.kernel,
      out_type=jax.ShapeDtypeStruct((_N, _D), jnp.float32),
      mesh=mesh,
      scratch_types=[
          pltpu.VMEM((_BATCH * _POS_PER_W,), jnp.int32),
          pltpu.VMEM((_CHUNK, _D), jnp.float32),
          pltpu.VMEM((_CHUNK, _D), jnp.float32),
          pltpu.VMEM((_CHUNK, _D), jnp.float32),
          pltpu.VMEM((_CHUNK, _D), jnp.float32),
          pltpu.VMEM((_CHUNK, _D), jnp.float32),
          pltpu.VMEM((_CHUNK, _D), jnp.float32),
          pltpu.SemaphoreType.DMA,
          pltpu.SemaphoreType.DMA,
          pltpu.SemaphoreType.DMA,
      ],
  )
  def emb(idx_hbm, table_hbm, pos_hbm, out_hbm, idx_v, tok0, tok1, tok2, tok3,
          pos0, pos1, sem_g, sem_p, sem_w):
    cid = lax.axis_index("c")
    sid = lax.axis_index("s")
    wid = sid * _NC + cid
    pstart = wid * _POS_PER_W
    tb = (tok0, tok1, tok2, tok3)
    pb = (pos0, pos1)

    def start_gather(b, pc, buf):
      pltpu.async_copy(
          table_hbm.at[
              idx_v.at[pl.ds(b * _POS_PER_W + pc * _CHUNK, _CHUNK)]
          ],
          tb[buf],
          sem_g,
      )

    def wait_gather(buf):
      pltpu.make_async_copy(
          table_hbm.at[idx_v.at[pl.ds(0, _CHUNK)]], tb[buf], sem_g
      ).wait()

    def start_pos(pc, buf):
      pltpu.async_copy(
          pos_hbm.at[pl.ds(pstart + pc * _CHUNK, _CHUNK)], pb[buf], sem_p
      )

    def wait_pos(buf):
      pltpu.make_async_copy(
          pos_hbm.at[pl.ds(0, _CHUNK)], pb[buf], sem_p
      ).wait()

    def start_write(b, pc, buf):
      pltpu.async_copy(
          tb[buf],
          out_hbm.at[pl.ds(b * _SEQ + pstart + pc * _CHUNK, _CHUNK)],
          sem_w,
      )

    def wait_write(buf):
      pltpu.make_async_copy(
          tb[buf], out_hbm.at[pl.ds(0, _CHUNK)], sem_w
      ).wait()

    def add_rows(p, q):
      tok = tb[p]
      pos = pb[q]

      def rows2(i, c2):
        r = i * 2
        for r2 in range(2):
          for g in range(_GROUPS):
            plsc.addupdate(
                tok.at[r + r2, pl.ds(g * 16, 16)],
                pos[r + r2, pl.ds(g * 16, 16)],
            )
        return c2

      lax.fori_loop(0, _CHUNK // 2, rows2, 0)

    for b in range(_BATCH):
      pltpu.sync_copy(
          idx_hbm.at[pl.ds(b * _SEQ + pstart, _POS_PER_W)],
          idx_v.at[pl.ds(b * _POS_PER_W, _POS_PER_W)],
      )

    start_pos(0, 0)
    start_gather(0, 0, 0)
    start_gather(1, 0, 1)

    def outer(it, carry):
      pcb = it * 2
      for k in range(8):
        b = k % 4
        p = k % 4
        q = k // 4
        pc = pcb + (k // 4)
        if k < 2:
          @pl.when(it > 0)
          def _():
            wait_write((k + 2) % 4)
        else:
          wait_write((k + 2) % 4)
        if k < 6:
          start_gather((k + 2) % 4, pcb + (k + 2) // 4, (k + 2) % 4)
        else:
          @pl.when(it < _OUTER - 1)
          def _():
            start_gather((k + 2) % 4, pcb + 2, (k + 2) % 4)
        if k == 2:
          start_pos(pcb + 1, 1)
        elif k == 6:
          @pl.when(it < _OUTER - 1)
          def _():
            start_pos(pcb + 2, 0)
        wait_gather(p)
        if b == 0:
          wait_pos(q)
        add_rows(p, q)
        start_write(b, pc, p)
      return carry

    lax.fori_loop(0, _OUTER, outer, 0)
    wait_write(2)
    wait_write(3)

  return emb


_emb = _make_emb_kernel()


@jax.jit
def kernel(x, tok_table, pos_table):
  idx = x.reshape(-1)
  out = _emb(idx, tok_table, pos_table)
  return out.reshape(x.shape[0], x.shape[1], _D)
